# Initial kernel scaffold; baseline (speedup 1.0000x reference)
import jax
import jax.numpy as jnp
from jax.experimental import pallas as pl


def kernel(x, hyperedge_index, W1, b1, g1, be1, W2, b2, Wmu, bmu, Wlv, blv, Wd1, bd1, Wd2, bd2, Wd3, bd3):
    raise NotImplementedError("write your pallas kernel here")


# speedup vs baseline: 12.7893x; 12.7893x over previous
import functools

import jax
import jax.numpy as jnp
from jax import lax
from jax.experimental import pallas as pl
from jax.experimental.pallas import tpu as pltpu
from jax.experimental.pallas import tpu_sc as plsc

_N = 10000
_E = 320000
_NP = 10240
_NSUB = 16
_RPT = _NP // _NSUB
_CHUNK = 128
_CH = -(-_E // (_NSUB * _CHUNK))
_EP = _NSUB * _CH * _CHUNK


def _sc_mesh():
    return plsc.VectorSubcoreMesh(core_axis_name="c", subcore_axis_name="s")


def _degrees_body(idx2_hbm, out_hbm, idx_v, val_v, acc_sh):
    c = lax.axis_index("c")
    s = lax.axis_index("s")
    r0 = s * _RPT
    pltpu.sync_copy(idx2_hbm.at[c, s], idx_v)

    def _zero(i, _):
        val_v[i, pl.ds(0, 16)] = jnp.zeros((16,), jnp.float32)
        return 0
    lax.fori_loop(0, _CHUNK, _zero, 0)
    for i in range(_RPT // _CHUNK):
        pltpu.sync_copy(val_v, acc_sh.at[pl.ds(r0 + i * _CHUNK, _CHUNK)])

    def _ones(i, _):
        val_v[i, pl.ds(0, 16)] = jnp.ones((16,), jnp.float32)
        return 0
    lax.fori_loop(0, _CHUNK, _ones, 0)
    plsc.subcore_barrier()

    def _step(j, _):
        pltpu.sync_copy(val_v, acc_sh.at[idx_v.at[j]], add=True)
        return 0
    lax.fori_loop(0, _CH, _step, 0)
    plsc.subcore_barrier()
    pltpu.sync_copy(acc_sh.at[pl.ds(r0, _RPT)], out_hbm.at[c, pl.ds(r0, _RPT)])


@functools.cache
def _degrees_kernel():
    return pl.kernel(
        _degrees_body,
        out_type=jax.ShapeDtypeStruct((2, _NP, 16), jnp.float32),
        mesh=_sc_mesh(),
        scratch_types=[
            pltpu.VMEM((_CH, _CHUNK), jnp.int32),
            pltpu.VMEM((_CHUNK, 16), jnp.float32),
            pltpu.VMEM_SHARED((_NP, 16), jnp.float32),
        ],
        compiler_params=pltpu.CompilerParams(use_tc_tiling_on_sc=False),
    )


def _pass_body(F, stage_src, src0_hbm, src1_hbm, gidx_hbm, sidx_hbm, out_hbm,
               gidx_v, sidx_v, rows_v, acc_sh, sem, src_sh=None):
    c = lax.axis_index("c")
    s = lax.axis_index("s")
    r0 = s * _RPT
    pltpu.sync_copy(gidx_hbm.at[s], gidx_v)
    pltpu.sync_copy(sidx_hbm.at[s], sidx_v)
    if stage_src:
        @pl.when(c == 0)
        def _():
            pltpu.sync_copy(src0_hbm.at[pl.ds(r0, _RPT)],
                            src_sh.at[pl.ds(r0, _RPT)])

        @pl.when(c == 1)
        def _():
            pltpu.sync_copy(src1_hbm.at[pl.ds(r0, _RPT)],
                            src_sh.at[pl.ds(r0, _RPT)])

    def _zero(i, _):
        for f in range(F // 16):
            rows_v[i, pl.ds(f * 16, 16)] = jnp.zeros((16,), jnp.float32)
        return 0
    lax.fori_loop(0, _CHUNK, _zero, 0)
    for i in range(_RPT // _CHUNK):
        pltpu.sync_copy(rows_v, acc_sh.at[pl.ds(r0 + i * _CHUNK, _CHUNK)])
    plsc.subcore_barrier()

    def _step(j, _):
        if stage_src:
            pltpu.async_copy(src_sh.at[gidx_v.at[j]], rows_v, sem).wait()
        else:
            @pl.when(c == 0)
            def _():
                pltpu.async_copy(src0_hbm.at[gidx_v.at[j]], rows_v, sem).wait()

            @pl.when(c == 1)
            def _():
                pltpu.async_copy(src1_hbm.at[gidx_v.at[j]], rows_v, sem).wait()
        pltpu.sync_copy(rows_v, acc_sh.at[sidx_v.at[j]], add=True)
        return 0
    lax.fori_loop(0, _CH, _step, 0)
    plsc.subcore_barrier()
    pltpu.sync_copy(acc_sh.at[pl.ds(r0, _RPT)], out_hbm.at[c, pl.ds(r0, _RPT)])


@functools.cache
def _make_pass(F, stage_src):
    scratch = [
        pltpu.VMEM((_CH, _CHUNK), jnp.int32),
        pltpu.VMEM((_CH, _CHUNK), jnp.int32),
        pltpu.VMEM((_CHUNK, F), jnp.float32),
        pltpu.VMEM_SHARED((_NP, F), jnp.float32),
        pltpu.SemaphoreType.DMA,
    ]
    if stage_src:
        scratch.append(pltpu.VMEM_SHARED((_NP, F), jnp.float32))
    return pl.kernel(
        functools.partial(_pass_body, F, stage_src),
        out_type=jax.ShapeDtypeStruct((2, _NP, F), jnp.float32),
        mesh=_sc_mesh(),
        scratch_types=scratch,
        compiler_params=pltpu.CompilerParams(use_tc_tiling_on_sc=False),
    )


def _mm(a, w):
    return lax.dot_general(a, w, (((1,), (1,)), ((), ())),
                           preferred_element_type=jnp.float32)


def _lrelu(t):
    return jnp.maximum(t, 0.01 * t)


_ZPAD32 = (_NP - _N, 32)
_ZPAD64 = (_NP - _N, 64)


def _t1_body(x_ref, w1_ref, deg_ref, xls_ref, dinv_ref, binv_ref):
    xl = _mm(x_ref[...], w1_ref[...])
    zp = jnp.zeros(_ZPAD32, jnp.float32)
    xls_ref[0, :, :] = jnp.concatenate([xl[:, :32], zp], axis=0)
    xls_ref[1, :, :] = jnp.concatenate([xl[:, 32:], zp], axis=0)
    dd = deg_ref[0, :, 0:1]
    bd = deg_ref[1, :, 0:1]
    dinv_ref[...] = jnp.where(dd > 0, 1.0 / dd, 0.0)
    binv_ref[...] = jnp.where(bd > 0, 1.0 / bd, 0.0)


def _scale_body(e_ref, binv_ref, o_ref):
    o_ref[...] = e_ref[...] * binv_ref[...][None]


def _t2_body(s_ref, dinv_ref, b1_ref, g1_ref, be1_ref, w2_ref, o_ref):
    h = jnp.concatenate([s_ref[0, :_N, :], s_ref[1, :_N, :]], axis=1)
    h = h * dinv_ref[: _N] + b1_ref[...]
    mu = jnp.mean(h, axis=1, keepdims=True)
    var = jnp.mean((h - mu) * (h - mu), axis=1, keepdims=True)
    h = (h - mu) * lax.rsqrt(var + 1e-5) * g1_ref[...] + be1_ref[...]
    h = _lrelu(h)
    xl = _mm(h, w2_ref[...])
    zp = jnp.zeros(_ZPAD32, jnp.float32)
    o_ref[0, :, :] = jnp.concatenate([xl[:, :32], zp], axis=0)
    o_ref[1, :, :] = jnp.concatenate([xl[:, 32:], zp], axis=0)


def _t3_body(s_ref, dinv_ref, b2_ref, wmu_ref, wlv_ref, o_ref):
    h = jnp.concatenate([s_ref[0, :_N, :], s_ref[1, :_N, :]], axis=1)
    h = _lrelu(h * dinv_ref[: _N] + b2_ref[...])
    zp = jnp.zeros(_ZPAD64, jnp.float32)
    o_ref[0, :, :] = jnp.concatenate([_mm(h, wmu_ref[...]), zp], axis=0)
    o_ref[1, :, :] = jnp.concatenate([_mm(h, wlv_ref[...]), zp], axis=0)


def _t4_body(s_ref, dinv_ref, bmu_ref, blv_ref, wd1_ref, bd1_ref,
             wd2_ref, bd2_ref, wd3_ref, bd3_ref, zd_ref, lv_ref):
    di = dinv_ref[: _N]
    mu = s_ref[0, :_N, :] * di + bmu_ref[...]
    lv_ref[...] = s_ref[1, :_N, :] * di + blv_ref[...]
    t = _lrelu(_mm(mu, wd1_ref[...]) + bd1_ref[...])
    t = _lrelu(_mm(t, wd2_ref[...]) + bd2_ref[...])
    d = jnp.sum(t * wd3_ref[...], axis=1, keepdims=True) + bd3_ref[0, 0]
    nrm = jnp.maximum(jnp.sqrt(jnp.sum(d * d)), 1e-8)
    zd_ref[...] = d / nrm


def _tc(body, out_shape):
    return pl.pallas_call(body, out_shape=out_shape)


def kernel(x, hyperedge_index, W1, b1, g1, be1, W2, b2, Wmu, bmu,
           Wlv, blv, Wd1, bd1, Wd2, bd2, Wd3, bd3):
    f32 = jnp.float32
    pad = jnp.full((_EP - _E,), _N, jnp.int32)
    ni = jnp.concatenate([hyperedge_index[0], pad]).reshape(_NSUB, _CH, _CHUNK)
    ei = jnp.concatenate([hyperedge_index[1], pad]).reshape(_NSUB, _CH, _CHUNK)
    idx2 = jnp.stack([ni, ei])

    b1r = b1.reshape(1, -1); g1r = g1.reshape(1, -1); be1r = be1.reshape(1, -1)
    b2r = b2.reshape(1, -1); bmur = bmu.reshape(1, -1); blvr = blv.reshape(1, -1)
    bd1r = bd1.reshape(1, -1); bd2r = bd2.reshape(1, -1); bd3r = bd3.reshape(1, -1)

    deg = _degrees_kernel()(idx2)
    _pass32 = _make_pass(32, True)
    _pass64 = _make_pass(64, False)

    sd = jax.ShapeDtypeStruct
    xls, dinv, binv = _tc(_t1_body, (sd((2, _NP, 32), f32),
                                     sd((_NP, 1), f32),
                                     sd((_NP, 1), f32)))(x, W1, deg)

    e1 = _pass32(xls[0], xls[1], ni, ei)
    ef1 = _tc(_scale_body, sd((2, _NP, 32), f32))(e1, binv)
    s1 = _pass32(ef1[0], ef1[1], ei, ni)

    xl2 = _tc(_t2_body, sd((2, _NP, 32), f32))(s1, dinv, b1r, g1r, be1r, W2)
    e2 = _pass32(xl2[0], xl2[1], ni, ei)
    ef2 = _tc(_scale_body, sd((2, _NP, 32), f32))(e2, binv)
    s2 = _pass32(ef2[0], ef2[1], ei, ni)

    xmv = _tc(_t3_body, sd((2, _NP, 64), f32))(s2, dinv, b2r, Wmu, Wlv)
    e3 = _pass64(xmv[0], xmv[1], ni, ei)
    ef3 = _tc(_scale_body, sd((2, _NP, 64), f32))(e3, binv)
    s3 = _pass64(ef3[0], ef3[1], ei, ni)

    zd, logvar = _tc(_t4_body, (sd((_N, 1), f32), sd((_N, 64), f32)))(
        s3, dinv, bmur, blvr, Wd1, bd1r, Wd2, bd2r, Wd3, bd3r)
    return (zd, zd, logvar)


# speedup vs baseline: 15.5377x; 1.2149x over previous
import functools

import jax
import jax.numpy as jnp
from jax import lax
from jax.experimental import pallas as pl
from jax.experimental.pallas import tpu as pltpu
from jax.experimental.pallas import tpu_sc as plsc

_N = 10000
_E = 320000
_NP = 10240
_NSUB = 16
_RPT = _NP // _NSUB
_CHUNK = 128
_CH = -(-_E // (_NSUB * _CHUNK))
_EP = _NSUB * _CH * _CHUNK


def _sc_mesh():
    return plsc.VectorSubcoreMesh(core_axis_name="c", subcore_axis_name="s")


def _degrees_body(idx2_hbm, out_hbm, idx_v, val_v, acc_sh):
    c = lax.axis_index("c")
    s = lax.axis_index("s")
    r0 = s * _RPT
    pltpu.sync_copy(idx2_hbm.at[c, s], idx_v)

    def _zero(i, _):
        val_v[i, pl.ds(0, 16)] = jnp.zeros((16,), jnp.float32)
        return 0
    lax.fori_loop(0, _CHUNK, _zero, 0)
    for i in range(_RPT // _CHUNK):
        pltpu.sync_copy(val_v, acc_sh.at[pl.ds(r0 + i * _CHUNK, _CHUNK)])

    def _ones(i, _):
        val_v[i, pl.ds(0, 16)] = jnp.ones((16,), jnp.float32)
        return 0
    lax.fori_loop(0, _CHUNK, _ones, 0)
    plsc.subcore_barrier()

    def _step(j, _):
        pltpu.sync_copy(val_v, acc_sh.at[idx_v.at[j]], add=True)
        return 0
    lax.fori_loop(0, _CH, _step, 0)
    plsc.subcore_barrier()
    pltpu.sync_copy(acc_sh.at[pl.ds(r0, _RPT)], out_hbm.at[c, pl.ds(r0, _RPT)])


@functools.cache
def _degrees_kernel():
    return pl.kernel(
        _degrees_body,
        out_type=jax.ShapeDtypeStruct((2, _NP, 16), jnp.float32),
        mesh=_sc_mesh(),
        scratch_types=[
            pltpu.VMEM((_CH, _CHUNK), jnp.int32),
            pltpu.VMEM((_CHUNK, 16), jnp.float32),
            pltpu.VMEM_SHARED((_NP, 16), jnp.float32),
        ],
        compiler_params=pltpu.CompilerParams(use_tc_tiling_on_sc=False),
    )


def _pass_body(F, stage_src, src0_hbm, src1_hbm, gidx_hbm, sidx_hbm, out_hbm,
               gidx_v, sidx_v, rows0_v, rows1_v, acc_sh, sem0, sem1,
               src_sh=None):
    c = lax.axis_index("c")
    s = lax.axis_index("s")
    r0 = s * _RPT
    pltpu.sync_copy(gidx_hbm.at[s], gidx_v)
    pltpu.sync_copy(sidx_hbm.at[s], sidx_v)
    if stage_src:
        @pl.when(c == 0)
        def _():
            pltpu.sync_copy(src0_hbm.at[pl.ds(r0, _RPT)],
                            src_sh.at[pl.ds(r0, _RPT)])

        @pl.when(c == 1)
        def _():
            pltpu.sync_copy(src1_hbm.at[pl.ds(r0, _RPT)],
                            src_sh.at[pl.ds(r0, _RPT)])

    def _zero(i, _):
        for f in range(F // 16):
            rows0_v[i, pl.ds(f * 16, 16)] = jnp.zeros((16,), jnp.float32)
        return 0
    lax.fori_loop(0, _CHUNK, _zero, 0)
    for i in range(_RPT // _CHUNK):
        pltpu.sync_copy(rows0_v, acc_sh.at[pl.ds(r0 + i * _CHUNK, _CHUNK)])
    plsc.subcore_barrier()

    def _fire(j, buf, sem):
        if stage_src:
            pltpu.async_copy(src_sh.at[gidx_v.at[j]], buf, sem)
        else:
            @pl.when(c == 0)
            def _():
                pltpu.async_copy(src0_hbm.at[gidx_v.at[j]], buf, sem)

            @pl.when(c == 1)
            def _():
                pltpu.async_copy(src1_hbm.at[gidx_v.at[j]], buf, sem)

    def _wait(buf, sem):
        pltpu.make_async_copy(src0_hbm.at[pl.ds(0, _CHUNK)], buf, sem).wait()

    def _scat(j, buf):
        pltpu.sync_copy(buf, acc_sh.at[sidx_v.at[j]], add=True)

    _fire(0, rows0_v, sem0)

    def _pair(jj, _):
        j = 2 * jj
        _wait(rows0_v, sem0)
        _fire(j + 1, rows1_v, sem1)
        _scat(j, rows0_v)
        _wait(rows1_v, sem1)
        _fire(j + 2, rows0_v, sem0)
        _scat(j + 1, rows1_v)
        return 0
    lax.fori_loop(0, (_CH - 1) // 2, _pair, 0)
    _wait(rows0_v, sem0)
    _scat(_CH - 1, rows0_v)
    plsc.subcore_barrier()
    pltpu.sync_copy(acc_sh.at[pl.ds(r0, _RPT)], out_hbm.at[c, pl.ds(r0, _RPT)])


@functools.cache
def _make_pass(F, stage_src):
    scratch = [
        pltpu.VMEM((_CH, _CHUNK), jnp.int32),
        pltpu.VMEM((_CH, _CHUNK), jnp.int32),
        pltpu.VMEM((_CHUNK, F), jnp.float32),
        pltpu.VMEM((_CHUNK, F), jnp.float32),
        pltpu.VMEM_SHARED((_NP, F), jnp.float32),
        pltpu.SemaphoreType.DMA,
        pltpu.SemaphoreType.DMA,
    ]
    if stage_src:
        scratch.append(pltpu.VMEM_SHARED((_NP, F), jnp.float32))
    return pl.kernel(
        functools.partial(_pass_body, F, stage_src),
        out_type=jax.ShapeDtypeStruct((2, _NP, F), jnp.float32),
        mesh=_sc_mesh(),
        scratch_types=scratch,
        compiler_params=pltpu.CompilerParams(use_tc_tiling_on_sc=False),
    )


def _mm(a, w):
    return lax.dot_general(a, w, (((1,), (1,)), ((), ())),
                           preferred_element_type=jnp.float32)


def _lrelu(t):
    return jnp.maximum(t, 0.01 * t)


_ZPAD32 = (_NP - _N, 32)
_ZPAD64 = (_NP - _N, 64)


def _t1_body(x_ref, w1_ref, deg_ref, xls_ref, dinv_ref, binv_ref):
    xl = _mm(x_ref[...], w1_ref[...])
    zp = jnp.zeros(_ZPAD32, jnp.float32)
    xls_ref[0, :, :] = jnp.concatenate([xl[:, :32], zp], axis=0)
    xls_ref[1, :, :] = jnp.concatenate([xl[:, 32:], zp], axis=0)
    dd = deg_ref[0, :, 0:1]
    bd = deg_ref[1, :, 0:1]
    dinv_ref[...] = jnp.where(dd > 0, 1.0 / dd, 0.0)
    binv_ref[...] = jnp.where(bd > 0, 1.0 / bd, 0.0)


def _scale_body(e_ref, binv_ref, o_ref):
    o_ref[...] = e_ref[...] * binv_ref[...][None]


def _t2_body(s_ref, dinv_ref, b1_ref, g1_ref, be1_ref, w2_ref, o_ref):
    h = jnp.concatenate([s_ref[0, :_N, :], s_ref[1, :_N, :]], axis=1)
    h = h * dinv_ref[: _N] + b1_ref[...]
    mu = jnp.mean(h, axis=1, keepdims=True)
    var = jnp.mean((h - mu) * (h - mu), axis=1, keepdims=True)
    h = (h - mu) * lax.rsqrt(var + 1e-5) * g1_ref[...] + be1_ref[...]
    h = _lrelu(h)
    xl = _mm(h, w2_ref[...])
    zp = jnp.zeros(_ZPAD32, jnp.float32)
    o_ref[0, :, :] = jnp.concatenate([xl[:, :32], zp], axis=0)
    o_ref[1, :, :] = jnp.concatenate([xl[:, 32:], zp], axis=0)


def _t3_body(s_ref, dinv_ref, b2_ref, wmu_ref, wlv_ref, o_ref):
    h = jnp.concatenate([s_ref[0, :_N, :], s_ref[1, :_N, :]], axis=1)
    h = _lrelu(h * dinv_ref[: _N] + b2_ref[...])
    zp = jnp.zeros(_ZPAD64, jnp.float32)
    o_ref[0, :, :] = jnp.concatenate([_mm(h, wmu_ref[...]), zp], axis=0)
    o_ref[1, :, :] = jnp.concatenate([_mm(h, wlv_ref[...]), zp], axis=0)


def _t4_body(s_ref, dinv_ref, bmu_ref, blv_ref, wd1_ref, bd1_ref,
             wd2_ref, bd2_ref, wd3_ref, bd3_ref, zd_ref, lv_ref):
    di = dinv_ref[: _N]
    mu = s_ref[0, :_N, :] * di + bmu_ref[...]
    lv_ref[...] = s_ref[1, :_N, :] * di + blv_ref[...]
    t = _lrelu(_mm(mu, wd1_ref[...]) + bd1_ref[...])
    t = _lrelu(_mm(t, wd2_ref[...]) + bd2_ref[...])
    d = jnp.sum(t * wd3_ref[...], axis=1, keepdims=True) + bd3_ref[0, 0]
    nrm = jnp.maximum(jnp.sqrt(jnp.sum(d * d)), 1e-8)
    zd_ref[...] = d / nrm


def _tc(body, out_shape):
    return pl.pallas_call(body, out_shape=out_shape)


def kernel(x, hyperedge_index, W1, b1, g1, be1, W2, b2, Wmu, bmu,
           Wlv, blv, Wd1, bd1, Wd2, bd2, Wd3, bd3):
    f32 = jnp.float32
    pad = jnp.full((_EP - _E,), _N, jnp.int32)
    ni = jnp.concatenate([hyperedge_index[0], pad]).reshape(_NSUB, _CH, _CHUNK)
    ei = jnp.concatenate([hyperedge_index[1], pad]).reshape(_NSUB, _CH, _CHUNK)
    idx2 = jnp.stack([ni, ei])

    b1r = b1.reshape(1, -1); g1r = g1.reshape(1, -1); be1r = be1.reshape(1, -1)
    b2r = b2.reshape(1, -1); bmur = bmu.reshape(1, -1); blvr = blv.reshape(1, -1)
    bd1r = bd1.reshape(1, -1); bd2r = bd2.reshape(1, -1); bd3r = bd3.reshape(1, -1)

    deg = _degrees_kernel()(idx2)
    _pass32 = _make_pass(32, True)
    _pass64 = _make_pass(64, False)

    sd = jax.ShapeDtypeStruct
    xls, dinv, binv = _tc(_t1_body, (sd((2, _NP, 32), f32),
                                     sd((_NP, 1), f32),
                                     sd((_NP, 1), f32)))(x, W1, deg)

    e1 = _pass32(xls[0], xls[1], ni, ei)
    ef1 = _tc(_scale_body, sd((2, _NP, 32), f32))(e1, binv)
    s1 = _pass32(ef1[0], ef1[1], ei, ni)

    xl2 = _tc(_t2_body, sd((2, _NP, 32), f32))(s1, dinv, b1r, g1r, be1r, W2)
    e2 = _pass32(xl2[0], xl2[1], ni, ei)
    ef2 = _tc(_scale_body, sd((2, _NP, 32), f32))(e2, binv)
    s2 = _pass32(ef2[0], ef2[1], ei, ni)

    xmv = _tc(_t3_body, sd((2, _NP, 64), f32))(s2, dinv, b2r, Wmu, Wlv)
    e3 = _pass64(xmv[0], xmv[1], ni, ei)
    ef3 = _tc(_scale_body, sd((2, _NP, 64), f32))(e3, binv)
    s3 = _pass64(ef3[0], ef3[1], ei, ni)

    zd, logvar = _tc(_t4_body, (sd((_N, 1), f32), sd((_N, 64), f32)))(
        s3, dinv, bmur, blvr, Wd1, bd1r, Wd2, bd2r, Wd3, bd3r)
    return (zd, zd, logvar)


# speedup vs baseline: 21.3146x; 1.3718x over previous
import functools

import jax
import jax.numpy as jnp
from jax import lax
from jax.experimental import pallas as pl
from jax.experimental.pallas import tpu as pltpu
from jax.experimental.pallas import tpu_sc as plsc

_N = 10000
_E = 320000
_NP = 10240
_NSUB = 16
_RPT = _NP // _NSUB
_CHUNK = 128
_CH = -(-_E // (_NSUB * _CHUNK))
_EP = _NSUB * _CH * _CHUNK


def _sc_mesh():
    return plsc.VectorSubcoreMesh(core_axis_name="c", subcore_axis_name="s")


def _degrees_body(idx2_hbm, out_hbm, idx_v, val_v, acc_sh):
    c = lax.axis_index("c")
    s = lax.axis_index("s")
    r0 = s * _RPT
    pltpu.sync_copy(idx2_hbm.at[c, s], idx_v)

    def _zero(i, _):
        val_v[i, pl.ds(0, 16)] = jnp.zeros((16,), jnp.float32)
        return 0
    lax.fori_loop(0, _CHUNK, _zero, 0)
    for i in range(_RPT // _CHUNK):
        pltpu.sync_copy(val_v, acc_sh.at[pl.ds(r0 + i * _CHUNK, _CHUNK)])

    def _ones(i, _):
        val_v[i, pl.ds(0, 16)] = jnp.ones((16,), jnp.float32)
        return 0
    lax.fori_loop(0, _CHUNK, _ones, 0)
    plsc.subcore_barrier()

    def _step(j, _):
        pltpu.sync_copy(val_v, acc_sh.at[idx_v.at[j]], add=True)
        return 0
    lax.fori_loop(0, _CH, _step, 0)
    plsc.subcore_barrier()
    pltpu.sync_copy(acc_sh.at[pl.ds(r0, _RPT)], out_hbm.at[c, pl.ds(r0, _RPT)])


@functools.cache
def _degrees_kernel():
    return pl.kernel(
        _degrees_body,
        out_type=jax.ShapeDtypeStruct((2, _NP, 16), jnp.float32),
        mesh=_sc_mesh(),
        scratch_types=[
            pltpu.VMEM((_CH, _CHUNK), jnp.int32),
            pltpu.VMEM((_CHUNK, 16), jnp.float32),
            pltpu.VMEM_SHARED((_NP, 16), jnp.float32),
        ],
        compiler_params=pltpu.CompilerParams(use_tc_tiling_on_sc=False),
    )


def _pass_body(F, scale_out, src0_hbm, src1_hbm, gidx_hbm, sidx_hbm, *rest):
    if scale_out:
        (binv_hbm, out_hbm, gidx_v, sidx_v, rows0_v, rows1_v, orow_v, binv_v,
         src_sh, acc_sh, sem0, sem1) = rest
    else:
        (out_hbm, gidx_v, sidx_v, rows0_v, rows1_v,
         src_sh, acc_sh, sem0, sem1) = rest
    c = lax.axis_index("c")
    s = lax.axis_index("s")
    r0 = s * _RPT
    pltpu.sync_copy(gidx_hbm.at[s], gidx_v)
    pltpu.sync_copy(sidx_hbm.at[s], sidx_v)
    if scale_out:
        pltpu.sync_copy(binv_hbm.at[pl.ds(r0, _RPT)], binv_v)

    @pl.when(c == 0)
    def _():
        pltpu.sync_copy(src0_hbm.at[pl.ds(r0, _RPT)],
                        src_sh.at[pl.ds(r0, _RPT)])

    @pl.when(c == 1)
    def _():
        pltpu.sync_copy(src1_hbm.at[pl.ds(r0, _RPT)],
                        src_sh.at[pl.ds(r0, _RPT)])

    def _zero(i, _):
        for f in range(F // 16):
            rows0_v[i, pl.ds(f * 16, 16)] = jnp.zeros((16,), jnp.float32)
        return 0
    lax.fori_loop(0, _CHUNK, _zero, 0)
    for i in range(_RPT // _CHUNK):
        pltpu.sync_copy(rows0_v, acc_sh.at[pl.ds(r0 + i * _CHUNK, _CHUNK)])
    plsc.subcore_barrier()

    def _fire(j, buf, sem):
        pltpu.async_copy(src_sh.at[gidx_v.at[j]], buf, sem)

    def _wait(buf, sem):
        pltpu.make_async_copy(src0_hbm.at[pl.ds(0, _CHUNK)], buf, sem).wait()

    def _scat(j, buf):
        pltpu.sync_copy(buf, acc_sh.at[sidx_v.at[j]], add=True)

    _fire(0, rows0_v, sem0)

    def _pair(jj, _):
        j = 2 * jj
        _wait(rows0_v, sem0)
        _fire(j + 1, rows1_v, sem1)
        _scat(j, rows0_v)
        _wait(rows1_v, sem1)
        _fire(j + 2, rows0_v, sem0)
        _scat(j + 1, rows1_v)
        return 0
    lax.fori_loop(0, (_CH - 1) // 2, _pair, 0)
    _wait(rows0_v, sem0)
    _scat(_CH - 1, rows0_v)
    plsc.subcore_barrier()
    if scale_out:
        pltpu.sync_copy(acc_sh.at[pl.ds(r0, _RPT)], orow_v)

        def _scale(g, _):
            bv = binv_v[pl.ds(g * 16, 16)]
            for k in range(16):
                i = g * 16 + k
                b = bv[k]
                for f in range(F // 16):
                    orow_v[i, pl.ds(f * 16, 16)] = (
                        orow_v[i, pl.ds(f * 16, 16)] * b)
            return 0
        lax.fori_loop(0, _RPT // 16, _scale, 0)
        pltpu.sync_copy(orow_v, out_hbm.at[c, pl.ds(r0, _RPT)])
    else:
        pltpu.sync_copy(acc_sh.at[pl.ds(r0, _RPT)],
                        out_hbm.at[c, pl.ds(r0, _RPT)])


@functools.cache
def _make_pass(F, scale_out):
    scratch = [
        pltpu.VMEM((_CH, _CHUNK), jnp.int32),
        pltpu.VMEM((_CH, _CHUNK), jnp.int32),
        pltpu.VMEM((_CHUNK, F), jnp.float32),
        pltpu.VMEM((_CHUNK, F), jnp.float32),
    ]
    if scale_out:
        scratch += [pltpu.VMEM((_RPT, F), jnp.float32),
                    pltpu.VMEM((_RPT,), jnp.float32)]
    scratch += [
        pltpu.VMEM_SHARED((_NP, F), jnp.float32),
        pltpu.VMEM_SHARED((_NP, F), jnp.float32),
        pltpu.SemaphoreType.DMA,
        pltpu.SemaphoreType.DMA,
    ]
    return pl.kernel(
        functools.partial(_pass_body, F, scale_out),
        out_type=jax.ShapeDtypeStruct((2, _NP, F), jnp.float32),
        mesh=_sc_mesh(),
        scratch_types=scratch,
        compiler_params=pltpu.CompilerParams(use_tc_tiling_on_sc=False),
    )


def _mm(a, w):
    return lax.dot_general(a, w, (((1,), (1,)), ((), ())),
                           preferred_element_type=jnp.float32)


def _lrelu(t):
    return jnp.maximum(t, 0.01 * t)


_ZPAD32 = (_NP - _N, 32)
_ZPAD64 = (_NP - _N, 64)


def _t1_body(x_ref, w1_ref, deg_ref, xls_ref, dinv_ref, binv_ref):
    xl = _mm(x_ref[...], w1_ref[...])
    zp = jnp.zeros(_ZPAD32, jnp.float32)
    xls_ref[0, :, :] = jnp.concatenate([xl[:, :32], zp], axis=0)
    xls_ref[1, :, :] = jnp.concatenate([xl[:, 32:], zp], axis=0)
    dd = deg_ref[0, :, 0:1]
    bd = deg_ref[1, :, 0]
    dinv_ref[...] = jnp.where(dd > 0, 1.0 / dd, 0.0)
    binv_ref[...] = jnp.where(bd > 0, 1.0 / bd, 0.0)


def _t2_body(s_ref, dinv_ref, b1_ref, g1_ref, be1_ref, w2_ref, o_ref):
    h = jnp.concatenate([s_ref[0, :_N, :], s_ref[1, :_N, :]], axis=1)
    h = h * dinv_ref[: _N] + b1_ref[...]
    mu = jnp.mean(h, axis=1, keepdims=True)
    var = jnp.mean((h - mu) * (h - mu), axis=1, keepdims=True)
    h = (h - mu) * lax.rsqrt(var + 1e-5) * g1_ref[...] + be1_ref[...]
    h = _lrelu(h)
    xl = _mm(h, w2_ref[...])
    zp = jnp.zeros(_ZPAD32, jnp.float32)
    o_ref[0, :, :] = jnp.concatenate([xl[:, :32], zp], axis=0)
    o_ref[1, :, :] = jnp.concatenate([xl[:, 32:], zp], axis=0)


def _t3_body(s_ref, dinv_ref, b2_ref, o_ref):
    h = jnp.concatenate([s_ref[0, :_N, :], s_ref[1, :_N, :]], axis=1)
    h = _lrelu(h * dinv_ref[: _N] + b2_ref[...])
    zp = jnp.zeros(_ZPAD32, jnp.float32)
    o_ref[0, :, :] = jnp.concatenate([h[:, :32], zp], axis=0)
    o_ref[1, :, :] = jnp.concatenate([h[:, 32:], zp], axis=0)


def _t4_body(s_ref, dinv_ref, wmu_ref, bmu_ref, wlv_ref, blv_ref,
             wd1_ref, bd1_ref, wd2_ref, bd2_ref, wd3_ref, bd3_ref,
             zd_ref, lv_ref):
    q = jnp.concatenate([s_ref[0, :_N, :], s_ref[1, :_N, :]], axis=1)
    q = q * dinv_ref[: _N]
    mu = _mm(q, wmu_ref[...]) + bmu_ref[...]
    lv_ref[...] = _mm(q, wlv_ref[...]) + blv_ref[...]
    t = _lrelu(_mm(mu, wd1_ref[...]) + bd1_ref[...])
    t = _lrelu(_mm(t, wd2_ref[...]) + bd2_ref[...])
    d = jnp.sum(t * wd3_ref[...], axis=1, keepdims=True) + bd3_ref[0, 0]
    nrm = jnp.maximum(jnp.sqrt(jnp.sum(d * d)), 1e-8)
    zd_ref[...] = d / nrm


def _tc(body, out_shape):
    return pl.pallas_call(body, out_shape=out_shape)


def kernel(x, hyperedge_index, W1, b1, g1, be1, W2, b2, Wmu, bmu,
           Wlv, blv, Wd1, bd1, Wd2, bd2, Wd3, bd3):
    f32 = jnp.float32
    pad = jnp.full((_EP - _E,), _N, jnp.int32)
    ni = jnp.concatenate([hyperedge_index[0], pad]).reshape(_NSUB, _CH, _CHUNK)
    ei = jnp.concatenate([hyperedge_index[1], pad]).reshape(_NSUB, _CH, _CHUNK)
    idx2 = jnp.stack([ni, ei])

    b1r = b1.reshape(1, -1); g1r = g1.reshape(1, -1); be1r = be1.reshape(1, -1)
    b2r = b2.reshape(1, -1); bmur = bmu.reshape(1, -1); blvr = blv.reshape(1, -1)
    bd1r = bd1.reshape(1, -1); bd2r = bd2.reshape(1, -1); bd3r = bd3.reshape(1, -1)

    deg = _degrees_kernel()(idx2)
    _pass_a = _make_pass(32, True)
    _pass_b = _make_pass(32, False)

    sd = jax.ShapeDtypeStruct
    xls, dinv, binv = _tc(_t1_body, (sd((2, _NP, 32), f32),
                                     sd((_NP, 1), f32),
                                     sd((_NP,), f32)))(x, W1, deg)

    e1 = _pass_a(xls[0], xls[1], ni, ei, binv)
    s1 = _pass_b(e1[0], e1[1], ei, ni)

    xl2 = _tc(_t2_body, sd((2, _NP, 32), f32))(s1, dinv, b1r, g1r, be1r, W2)
    e2 = _pass_a(xl2[0], xl2[1], ni, ei, binv)
    s2 = _pass_b(e2[0], e2[1], ei, ni)

    h2s = _tc(_t3_body, sd((2, _NP, 32), f32))(s2, dinv, b2r)
    e3 = _pass_a(h2s[0], h2s[1], ni, ei, binv)
    s3 = _pass_b(e3[0], e3[1], ei, ni)

    zd, logvar = _tc(_t4_body, (sd((_N, 1), f32), sd((_N, 64), f32)))(
        s3, dinv, Wmu, bmur, Wlv, blvr, Wd1, bd1r, Wd2, bd2r, Wd3, bd3r)
    return (zd, zd, logvar)


# speedup vs baseline: 24.1344x; 1.1323x over previous
import functools

import jax
import jax.numpy as jnp
from jax import lax
from jax.experimental import pallas as pl
from jax.experimental.pallas import tpu as pltpu
from jax.experimental.pallas import tpu_sc as plsc

_N = 10000
_E = 320000
_NP = 10240
_NSUB = 16
_RPT = _NP // _NSUB
_CHUNK = 128
_CH = -(-_E // (_NSUB * _CHUNK))
_EP = _NSUB * _CH * _CHUNK


def _sc_mesh():
    return plsc.VectorSubcoreMesh(core_axis_name="c", subcore_axis_name="s")


def _degrees_body(idx2_hbm, out_hbm, idx_v, val_v, acc_sh):
    c = lax.axis_index("c")
    s = lax.axis_index("s")
    r0 = s * _RPT
    pltpu.sync_copy(idx2_hbm.at[c, s], idx_v)

    def _zero(i, _):
        val_v[i, pl.ds(0, 16)] = jnp.zeros((16,), jnp.float32)
        return 0
    lax.fori_loop(0, _CHUNK, _zero, 0)
    for i in range(_RPT // _CHUNK):
        pltpu.sync_copy(val_v, acc_sh.at[pl.ds(r0 + i * _CHUNK, _CHUNK)])

    def _ones(i, _):
        val_v[i, pl.ds(0, 16)] = jnp.ones((16,), jnp.float32)
        return 0
    lax.fori_loop(0, _CHUNK, _ones, 0)
    plsc.subcore_barrier()

    def _step(j, _):
        pltpu.sync_copy(val_v, acc_sh.at[idx_v.at[j]], add=True)
        return 0
    lax.fori_loop(0, _CH, _step, 0)
    plsc.subcore_barrier()
    pltpu.sync_copy(acc_sh.at[pl.ds(r0, _RPT)], out_hbm.at[c, pl.ds(r0, _RPT)])


@functools.cache
def _degrees_kernel():
    return pl.kernel(
        _degrees_body,
        out_type=jax.ShapeDtypeStruct((2, _NP, 16), jnp.float32),
        mesh=_sc_mesh(),
        scratch_types=[
            pltpu.VMEM((_CH, _CHUNK), jnp.int32),
            pltpu.VMEM((_CHUNK, 16), jnp.float32),
            pltpu.VMEM_SHARED((_NP, 16), jnp.float32),
        ],
        compiler_params=pltpu.CompilerParams(use_tc_tiling_on_sc=False),
    )


def _conv_body(src0_hbm, src1_hbm, gidx_hbm, sidx_hbm, binv_hbm, out_hbm,
               gidx_v, sidx_v, rows0_v, rows1_v, orow_v, binv_v,
               src_sh, acc_sh, sem0, sem1):
    F = 32
    c = lax.axis_index("c")
    s = lax.axis_index("s")
    r0 = s * _RPT
    pltpu.sync_copy(gidx_hbm.at[s], gidx_v)
    pltpu.sync_copy(sidx_hbm.at[s], sidx_v)
    pltpu.sync_copy(binv_hbm.at[pl.ds(r0, _RPT)], binv_v)

    @pl.when(c == 0)
    def _():
        pltpu.sync_copy(src0_hbm.at[pl.ds(r0, _RPT)],
                        src_sh.at[pl.ds(r0, _RPT)])

    @pl.when(c == 1)
    def _():
        pltpu.sync_copy(src1_hbm.at[pl.ds(r0, _RPT)],
                        src_sh.at[pl.ds(r0, _RPT)])

    def _zero_buf(buf):
        def _z(i, _):
            for f in range(F // 16):
                buf[i, pl.ds(f * 16, 16)] = jnp.zeros((16,), jnp.float32)
            return 0
        lax.fori_loop(0, _CHUNK, _z, 0)

    def _zero_rows(dst_sh, buf):
        for i in range(_RPT // _CHUNK):
            pltpu.sync_copy(buf, dst_sh.at[pl.ds(r0 + i * _CHUNK, _CHUNK)])

    def _wait(buf, sem):
        pltpu.make_async_copy(src0_hbm.at[pl.ds(0, _CHUNK)], buf, sem).wait()

    def _seg(from_sh, to_sh, gv, sv):
        def _fire(j, buf, sem):
            pltpu.async_copy(from_sh.at[gv.at[j]], buf, sem)

        def _scat(j, buf):
            pltpu.sync_copy(buf, to_sh.at[sv.at[j]], add=True)

        _fire(0, rows0_v, sem0)

        def _pair(jj, _):
            j = 2 * jj
            _wait(rows0_v, sem0)
            _fire(j + 1, rows1_v, sem1)
            _scat(j, rows0_v)
            _wait(rows1_v, sem1)
            _fire(j + 2, rows0_v, sem0)
            _scat(j + 1, rows1_v)
            return 0
        lax.fori_loop(0, (_CH - 1) // 2, _pair, 0)
        _wait(rows0_v, sem0)
        _scat(_CH - 1, rows0_v)

    _zero_buf(rows0_v)
    _zero_rows(acc_sh, rows0_v)
    plsc.subcore_barrier()

    _seg(src_sh, acc_sh, gidx_v, sidx_v)
    plsc.subcore_barrier()

    pltpu.sync_copy(acc_sh.at[pl.ds(r0, _RPT)], orow_v)

    def _scale(g, _):
        bv = binv_v[pl.ds(g * 16, 16)]
        for k in range(16):
            i = g * 16 + k
            b = bv[k]
            for f in range(F // 16):
                orow_v[i, pl.ds(f * 16, 16)] = (
                    orow_v[i, pl.ds(f * 16, 16)] * b)
        return 0
    lax.fori_loop(0, _RPT // 16, _scale, 0)
    pltpu.sync_copy(orow_v, acc_sh.at[pl.ds(r0, _RPT)])
    _zero_buf(rows0_v)
    _zero_rows(src_sh, rows0_v)
    plsc.subcore_barrier()

    _seg(acc_sh, src_sh, sidx_v, gidx_v)
    plsc.subcore_barrier()
    pltpu.sync_copy(src_sh.at[pl.ds(r0, _RPT)], out_hbm.at[c, pl.ds(r0, _RPT)])


@functools.cache
def _conv_kernel():
    F = 32
    return pl.kernel(
        _conv_body,
        out_type=jax.ShapeDtypeStruct((2, _NP, F), jnp.float32),
        mesh=_sc_mesh(),
        scratch_types=[
            pltpu.VMEM((_CH, _CHUNK), jnp.int32),
            pltpu.VMEM((_CH, _CHUNK), jnp.int32),
            pltpu.VMEM((_CHUNK, F), jnp.float32),
            pltpu.VMEM((_CHUNK, F), jnp.float32),
            pltpu.VMEM((_RPT, F), jnp.float32),
            pltpu.VMEM((_RPT,), jnp.float32),
            pltpu.VMEM_SHARED((_NP, F), jnp.float32),
            pltpu.VMEM_SHARED((_NP, F), jnp.float32),
            pltpu.SemaphoreType.DMA,
            pltpu.SemaphoreType.DMA,
        ],
        compiler_params=pltpu.CompilerParams(use_tc_tiling_on_sc=False),
    )


def _mm(a, w):
    return lax.dot_general(a, w, (((1,), (1,)), ((), ())),
                           preferred_element_type=jnp.float32)


def _lrelu(t):
    return jnp.maximum(t, 0.01 * t)


_ZPAD32 = (_NP - _N, 32)
_ZPAD64 = (_NP - _N, 64)


def _t1_body(x_ref, w1_ref, deg_ref, xls_ref, dinv_ref, binv_ref):
    xl = _mm(x_ref[...], w1_ref[...])
    zp = jnp.zeros(_ZPAD32, jnp.float32)
    xls_ref[0, :, :] = jnp.concatenate([xl[:, :32], zp], axis=0)
    xls_ref[1, :, :] = jnp.concatenate([xl[:, 32:], zp], axis=0)
    dd = deg_ref[0, :, 0:1]
    bd = deg_ref[1, :, 0]
    dinv_ref[...] = jnp.where(dd > 0, 1.0 / dd, 0.0)
    binv_ref[...] = jnp.where(bd > 0, 1.0 / bd, 0.0)


def _t2_body(s_ref, dinv_ref, b1_ref, g1_ref, be1_ref, w2_ref, o_ref):
    h = jnp.concatenate([s_ref[0, :_N, :], s_ref[1, :_N, :]], axis=1)
    h = h * dinv_ref[: _N] + b1_ref[...]
    mu = jnp.mean(h, axis=1, keepdims=True)
    var = jnp.mean((h - mu) * (h - mu), axis=1, keepdims=True)
    h = (h - mu) * lax.rsqrt(var + 1e-5) * g1_ref[...] + be1_ref[...]
    h = _lrelu(h)
    xl = _mm(h, w2_ref[...])
    zp = jnp.zeros(_ZPAD32, jnp.float32)
    o_ref[0, :, :] = jnp.concatenate([xl[:, :32], zp], axis=0)
    o_ref[1, :, :] = jnp.concatenate([xl[:, 32:], zp], axis=0)


def _t3_body(s_ref, dinv_ref, b2_ref, o_ref):
    h = jnp.concatenate([s_ref[0, :_N, :], s_ref[1, :_N, :]], axis=1)
    h = _lrelu(h * dinv_ref[: _N] + b2_ref[...])
    zp = jnp.zeros(_ZPAD32, jnp.float32)
    o_ref[0, :, :] = jnp.concatenate([h[:, :32], zp], axis=0)
    o_ref[1, :, :] = jnp.concatenate([h[:, 32:], zp], axis=0)


def _t4_body(s_ref, dinv_ref, wmu_ref, bmu_ref, wlv_ref, blv_ref,
             wd1_ref, bd1_ref, wd2_ref, bd2_ref, wd3_ref, bd3_ref,
             zd_ref, lv_ref):
    q = jnp.concatenate([s_ref[0, :_N, :], s_ref[1, :_N, :]], axis=1)
    q = q * dinv_ref[: _N]
    mu = _mm(q, wmu_ref[...]) + bmu_ref[...]
    lv_ref[...] = _mm(q, wlv_ref[...]) + blv_ref[...]
    t = _lrelu(_mm(mu, wd1_ref[...]) + bd1_ref[...])
    t = _lrelu(_mm(t, wd2_ref[...]) + bd2_ref[...])
    d = jnp.sum(t * wd3_ref[...], axis=1, keepdims=True) + bd3_ref[0, 0]
    nrm = jnp.maximum(jnp.sqrt(jnp.sum(d * d)), 1e-8)
    zd_ref[...] = d / nrm


def _tc(body, out_shape):
    return pl.pallas_call(body, out_shape=out_shape)


def kernel(x, hyperedge_index, W1, b1, g1, be1, W2, b2, Wmu, bmu,
           Wlv, blv, Wd1, bd1, Wd2, bd2, Wd3, bd3):
    f32 = jnp.float32
    pad = jnp.full((_EP - _E,), _N, jnp.int32)
    ni = jnp.concatenate([hyperedge_index[0], pad]).reshape(_NSUB, _CH, _CHUNK)
    ei = jnp.concatenate([hyperedge_index[1], pad]).reshape(_NSUB, _CH, _CHUNK)
    idx2 = jnp.stack([ni, ei])

    b1r = b1.reshape(1, -1); g1r = g1.reshape(1, -1); be1r = be1.reshape(1, -1)
    b2r = b2.reshape(1, -1); bmur = bmu.reshape(1, -1); blvr = blv.reshape(1, -1)
    bd1r = bd1.reshape(1, -1); bd2r = bd2.reshape(1, -1); bd3r = bd3.reshape(1, -1)

    deg = _degrees_kernel()(idx2)
    _conv = _conv_kernel()

    sd = jax.ShapeDtypeStruct
    xls, dinv, binv = _tc(_t1_body, (sd((2, _NP, 32), f32),
                                     sd((_NP, 1), f32),
                                     sd((_NP,), f32)))(x, W1, deg)

    s1 = _conv(xls[0], xls[1], ni, ei, binv)
    xl2 = _tc(_t2_body, sd((2, _NP, 32), f32))(s1, dinv, b1r, g1r, be1r, W2)
    s2 = _conv(xl2[0], xl2[1], ni, ei, binv)
    h2s = _tc(_t3_body, sd((2, _NP, 32), f32))(s2, dinv, b2r)
    s3 = _conv(h2s[0], h2s[1], ni, ei, binv)

    zd, logvar = _tc(_t4_body, (sd((_N, 1), f32), sd((_N, 64), f32)))(
        s3, dinv, Wmu, bmur, Wlv, blvr, Wd1, bd1r, Wd2, bd2r, Wd3, bd3r)
    return (zd, zd, logvar)
